# baseline (speedup 1.0000x reference)
import functools

import jax
import jax.numpy as jnp
from jax import lax
from jax.experimental import pallas as pl
from jax.experimental.pallas import tpu as pltpu
from jax.experimental.pallas import tpu_sc as plsc

N_SRC = 10000
N_DST = 2048
E = 320000
D = 128

NC = 2
NS = 16
NW = NC * NS
EDGES_PER_TILE = E // NW
CHUNK = 128
NCHUNK = -(-EDGES_PER_TILE // CHUNK)
SRC_ROWS = NCHUNK + 8
N_ACC = N_DST + NS
ROWS_PER_TILE = N_DST // NS


def _sc_aggregate(x2, src_p, dst_p, z_acc):
    mesh = plsc.VectorSubcoreMesh(core_axis_name="c", subcore_axis_name="s")

    @functools.partial(
        pl.kernel,
        mesh=mesh,
        out_type=jax.ShapeDtypeStruct((NC, N_DST, 2, D), jnp.float32),
        scratch_types=[
            pltpu.VMEM((SRC_ROWS, CHUNK), jnp.int32),
            pltpu.VMEM((SRC_ROWS, CHUNK), jnp.int32),
            pltpu.VMEM((CHUNK, 2, D), jnp.float32),
            pltpu.VMEM((CHUNK, 2, D), jnp.float32),
            pltpu.VMEM_SHARED((N_ACC, 2, D), jnp.float32),
            pltpu.SemaphoreType.DMA,
            pltpu.SemaphoreType.DMA,
        ],
    )
    def k(x2_hbm, src_hbm, dst_hbm, zacc_hbm, acc_out,
          src_v, dst_v, rows_a, rows_b, acc_sh, sem_ga, sem_gb):
        cid = lax.axis_index("c")
        sid = lax.axis_index("s")
        wid = cid * NS + sid
        row0 = sid * ROWS_PER_TILE

        pltpu.sync_copy(src_hbm.at[wid], src_v)
        pltpu.sync_copy(dst_hbm.at[wid], dst_v)

        pltpu.sync_copy(zacc_hbm.at[pl.ds(row0, ROWS_PER_TILE)],
                        acc_sh.at[pl.ds(row0, ROWS_PER_TILE)])

        plsc.subcore_barrier()

        pltpu.async_copy(x2_hbm.at[src_v.at[0]], rows_a, sem_ga)

        def body(i, carry):
            c0 = 2 * i
            c1 = 2 * i + 1
            pltpu.make_async_copy(x2_hbm.at[src_v.at[c0]], rows_a,
                                  sem_ga).wait()
            pltpu.async_copy(x2_hbm.at[src_v.at[c1]], rows_b, sem_gb)
            pltpu.sync_copy(rows_a, acc_sh.at[dst_v.at[c0]], add=True)
            pltpu.make_async_copy(x2_hbm.at[src_v.at[c1]], rows_b,
                                  sem_gb).wait()
            pltpu.async_copy(x2_hbm.at[src_v.at[c1 + 1]], rows_a, sem_ga)
            pltpu.sync_copy(rows_b, acc_sh.at[dst_v.at[c1]], add=True)
            return carry

        lax.fori_loop(0, (NCHUNK - 1) // 2, body, 0)

        pltpu.make_async_copy(x2_hbm.at[src_v.at[NCHUNK - 1]], rows_a,
                              sem_ga).wait()
        pltpu.sync_copy(rows_a, acc_sh.at[dst_v.at[NCHUNK - 1]], add=True)

        plsc.subcore_barrier()

        pltpu.sync_copy(acc_sh.at[pl.ds(row0, ROWS_PER_TILE)],
                        acc_out.at[cid, pl.ds(row0, ROWS_PER_TILE)])

    return k(x2, src_p, dst_p, z_acc)


def _tc_body(acc_ref, xt_ref, wlt_ref, wrt_ref, b_ref, o_ref):
    comb = acc_ref[0] + acc_ref[1]
    acc = comb[:, 0, :]
    cnt = comb[:, 1, 0:1]
    agg = acc / jnp.maximum(cnt, 1.0)
    out = (jnp.dot(agg, wlt_ref[...], preferred_element_type=jnp.float32)
           + b_ref[...]
           + jnp.dot(xt_ref[...], wrt_ref[...],
                     preferred_element_type=jnp.float32))
    norm = jnp.sqrt(jnp.sum(out * out, axis=1, keepdims=True))
    o_ref[...] = out / jnp.maximum(norm, 1e-12)


def kernel(x, edge_index, W_l, W_r, b):
    src = edge_index[0].astype(jnp.int32)
    dst = edge_index[1].astype(jnp.int32)
    x2 = jnp.stack([x, jnp.ones_like(x)], axis=1)
    src_p = jnp.pad(src.reshape(NW, EDGES_PER_TILE),
                    ((0, 0), (0, SRC_ROWS * CHUNK - EDGES_PER_TILE))
                    ).reshape(NW, SRC_ROWS, CHUNK)
    npad = SRC_ROWS * CHUNK - EDGES_PER_TILE
    junk = jnp.broadcast_to(
        (N_DST + jnp.arange(NW, dtype=jnp.int32) % NS)[:, None], (NW, npad))
    dst_p = jnp.concatenate(
        [dst.reshape(NW, EDGES_PER_TILE), junk],
        axis=1).reshape(NW, SRC_ROWS, CHUNK)
    z_acc = jnp.zeros((N_DST, 2, D), dtype=jnp.float32)

    acc_p = _sc_aggregate(x2, src_p, dst_p, z_acc)

    out = pl.pallas_call(
        _tc_body,
        out_shape=jax.ShapeDtypeStruct((N_DST, D), jnp.float32),
    )(acc_p, x[:N_DST], W_l.T, W_r.T, b.reshape(1, D))
    return out


# speedup vs baseline: 1.2179x; 1.2179x over previous
import functools

import jax
import jax.numpy as jnp
from jax import lax
from jax.experimental import pallas as pl
from jax.experimental.pallas import tpu as pltpu
from jax.experimental.pallas import tpu_sc as plsc

N_SRC = 10000
N_DST = 2048
E = 320000
D = 128

NC = 2
NS = 16
NW = NC * NS
EDGES_PER_TILE = E // NW
CHUNK = 128
NCHUNK = -(-EDGES_PER_TILE // CHUNK)
NCHUNK_PAD = NCHUNK + 1
SRC_ROWS = NCHUNK_PAD + 8
N_ACC = N_DST + NS
ROWS_PER_TILE = N_DST // NS


def _sc_aggregate(x, src_p, dst_p, z_acc):
    mesh = plsc.VectorSubcoreMesh(core_axis_name="c", subcore_axis_name="s")

    @functools.partial(
        pl.kernel,
        mesh=mesh,
        out_type=(
            jax.ShapeDtypeStruct((NC, N_DST, D), jnp.float32),
            jax.ShapeDtypeStruct((NC, N_DST, D), jnp.float32),
        ),
        scratch_types=[
            pltpu.VMEM((SRC_ROWS, CHUNK), jnp.int32),
            pltpu.VMEM((NCHUNK_PAD, CHUNK), jnp.int32),
            pltpu.VMEM((CHUNK, D), jnp.float32),
            pltpu.VMEM((CHUNK, D), jnp.float32),
            pltpu.VMEM((CHUNK, D), jnp.float32),
            pltpu.VMEM_SHARED((N_ACC, D), jnp.float32),
            pltpu.VMEM_SHARED((N_ACC, D), jnp.float32),
            pltpu.SemaphoreType.DMA,
            pltpu.SemaphoreType.DMA,
            pltpu.SemaphoreType.DMA,
            pltpu.SemaphoreType.DMA,
            pltpu.SemaphoreType.DMA,
        ],
    )
    def k(x_hbm, src_hbm, dst_hbm, zacc_hbm, acc_out, cnt_out,
          src_v, dst_v, rows_a, rows_b, ones_v, acc_sh, cnt_sh,
          sem_ga, sem_gb, sem_sa, sem_sb, sem_c):
        cid = lax.axis_index("c")
        sid = lax.axis_index("s")
        wid = cid * NS + sid
        row0 = sid * ROWS_PER_TILE

        pltpu.sync_copy(src_hbm.at[wid], src_v)
        pltpu.sync_copy(dst_hbm.at[wid], dst_v)

        pltpu.sync_copy(zacc_hbm.at[pl.ds(row0, ROWS_PER_TILE)],
                        acc_sh.at[pl.ds(row0, ROWS_PER_TILE)])
        pltpu.sync_copy(zacc_hbm.at[pl.ds(row0, ROWS_PER_TILE)],
                        cnt_sh.at[pl.ds(row0, ROWS_PER_TILE)])

        one16 = jnp.full((16,), 1.0, dtype=jnp.float32)

        def fill_ones(r, carry):
            for j in range(D // 16):
                ones_v[r, pl.ds(j * 16, 16)] = one16
            return carry

        lax.fori_loop(0, CHUNK, fill_ones, 0)

        plsc.subcore_barrier()

        pltpu.async_copy(x_hbm.at[src_v.at[0]], rows_a, sem_ga)

        def body(i, carry):
            c0 = 2 * i
            c1 = 2 * i + 1
            pltpu.make_async_copy(x_hbm.at[src_v.at[c0]], rows_a,
                                  sem_ga).wait()
            pltpu.async_copy(x_hbm.at[src_v.at[c1]], rows_b, sem_gb)
            pltpu.async_copy(rows_a, acc_sh.at[dst_v.at[c0]], sem_sa,
                             add=True)
            pltpu.async_copy(ones_v, cnt_sh.at[dst_v.at[c0]], sem_c,
                             add=True)
            pltpu.make_async_copy(rows_a, acc_sh.at[dst_v.at[c0]],
                                  sem_sa).wait()
            pltpu.make_async_copy(ones_v, cnt_sh.at[dst_v.at[c0]],
                                  sem_c).wait()
            pltpu.make_async_copy(x_hbm.at[src_v.at[c1]], rows_b,
                                  sem_gb).wait()
            pltpu.async_copy(x_hbm.at[src_v.at[c1 + 1]], rows_a, sem_ga)
            pltpu.async_copy(rows_b, acc_sh.at[dst_v.at[c1]], sem_sb,
                             add=True)
            pltpu.async_copy(ones_v, cnt_sh.at[dst_v.at[c1]], sem_c,
                             add=True)
            pltpu.make_async_copy(rows_b, acc_sh.at[dst_v.at[c1]],
                                  sem_sb).wait()
            pltpu.make_async_copy(ones_v, cnt_sh.at[dst_v.at[c1]],
                                  sem_c).wait()
            return carry

        lax.fori_loop(0, (NCHUNK - 1) // 2, body, 0)

        pltpu.make_async_copy(x_hbm.at[src_v.at[NCHUNK - 1]], rows_a,
                              sem_ga).wait()
        pltpu.sync_copy(rows_a, acc_sh.at[dst_v.at[NCHUNK - 1]], add=True)
        pltpu.sync_copy(ones_v, cnt_sh.at[dst_v.at[NCHUNK - 1]], add=True)

        plsc.subcore_barrier()

        pltpu.sync_copy(acc_sh.at[pl.ds(row0, ROWS_PER_TILE)],
                        acc_out.at[cid, pl.ds(row0, ROWS_PER_TILE)])
        pltpu.sync_copy(cnt_sh.at[pl.ds(row0, ROWS_PER_TILE)],
                        cnt_out.at[cid, pl.ds(row0, ROWS_PER_TILE)])

    return k(x, src_p, dst_p, z_acc)


def _tc_body(acc_ref, cnt_ref, xt_ref, wlt_ref, wrt_ref, b_ref, o_ref):
    acc = acc_ref[0] + acc_ref[1]
    cnt = (cnt_ref[0] + cnt_ref[1])[:, None]
    agg = acc / jnp.maximum(cnt, 1.0)
    out = (jnp.dot(agg, wlt_ref[...], preferred_element_type=jnp.float32)
           + b_ref[...]
           + jnp.dot(xt_ref[...], wrt_ref[...],
                     preferred_element_type=jnp.float32))
    norm = jnp.sqrt(jnp.sum(out * out, axis=1, keepdims=True))
    o_ref[...] = out / jnp.maximum(norm, 1e-12)


def kernel(x, edge_index, W_l, W_r, b):
    src = edge_index[0].astype(jnp.int32)
    dst = edge_index[1].astype(jnp.int32)
    src_p = jnp.pad(src.reshape(NW, EDGES_PER_TILE),
                    ((0, 0), (0, SRC_ROWS * CHUNK - EDGES_PER_TILE))
                    ).reshape(NW, SRC_ROWS, CHUNK)
    npad = NCHUNK_PAD * CHUNK - EDGES_PER_TILE
    junk = jnp.broadcast_to(
        (N_DST + jnp.arange(NW, dtype=jnp.int32) % NS)[:, None], (NW, npad))
    dst_p = jnp.concatenate(
        [dst.reshape(NW, EDGES_PER_TILE), junk],
        axis=1).reshape(NW, NCHUNK_PAD, CHUNK)
    z_acc = jnp.zeros((N_DST, D), dtype=jnp.float32)

    acc_p, cnt_p = _sc_aggregate(x, src_p, dst_p, z_acc)

    out = pl.pallas_call(
        _tc_body,
        out_shape=jax.ShapeDtypeStruct((N_DST, D), jnp.float32),
    )(acc_p, cnt_p[:, :, 0], x[:N_DST], W_l.T, W_r.T, b.reshape(1, D))
    return out


# speedup vs baseline: 1.3460x; 1.1052x over previous
import functools

import jax
import jax.numpy as jnp
from jax import lax
from jax.experimental import pallas as pl
from jax.experimental.pallas import tpu as pltpu
from jax.experimental.pallas import tpu_sc as plsc

N_SRC = 10000
N_DST = 2048
E = 320000
D = 128

NC = 2
NS = 16
NW = NC * NS
EDGES_PER_TILE = E // NW
CHUNK = 128
NCHUNK = -(-EDGES_PER_TILE // CHUNK)
NCHUNK_PAD = NCHUNK + 1
SRC_ROWS = NCHUNK_PAD + 8
N_ACC = N_DST + NS
ROWS_PER_TILE = N_DST // NS


def _sc_aggregate(x, src_p, dst_p, z_acc):
    mesh = plsc.VectorSubcoreMesh(core_axis_name="c", subcore_axis_name="s")

    @functools.partial(
        pl.kernel,
        mesh=mesh,
        out_type=jax.ShapeDtypeStruct((NC, N_DST, D), jnp.float32),
        scratch_types=[
            pltpu.VMEM((SRC_ROWS, CHUNK), jnp.int32),
            pltpu.VMEM((NCHUNK_PAD, CHUNK), jnp.int32),
            pltpu.VMEM((CHUNK, D), jnp.float32),
            pltpu.VMEM((CHUNK, D), jnp.float32),
            pltpu.VMEM_SHARED((N_ACC, D), jnp.float32),
            pltpu.SemaphoreType.DMA,
            pltpu.SemaphoreType.DMA,
        ],
    )
    def k(x_hbm, src_hbm, dst_hbm, zacc_hbm, acc_out,
          src_v, dst_v, rows_a, rows_b, acc_sh, sem_ga, sem_gb):
        cid = lax.axis_index("c")
        sid = lax.axis_index("s")
        wid = cid * NS + sid
        row0 = sid * ROWS_PER_TILE

        pltpu.sync_copy(src_hbm.at[wid], src_v)
        pltpu.sync_copy(dst_hbm.at[wid], dst_v)

        pltpu.sync_copy(zacc_hbm.at[pl.ds(row0, ROWS_PER_TILE)],
                        acc_sh.at[pl.ds(row0, ROWS_PER_TILE)])

        plsc.subcore_barrier()

        pltpu.async_copy(x_hbm.at[src_v.at[0]], rows_a, sem_ga)

        def body(i, carry):
            c0 = 2 * i
            c1 = 2 * i + 1
            pltpu.make_async_copy(x_hbm.at[src_v.at[c0]], rows_a,
                                  sem_ga).wait()
            pltpu.async_copy(x_hbm.at[src_v.at[c1]], rows_b, sem_gb)
            pltpu.sync_copy(rows_a, acc_sh.at[dst_v.at[c0]], add=True)
            pltpu.make_async_copy(x_hbm.at[src_v.at[c1]], rows_b,
                                  sem_gb).wait()
            pltpu.async_copy(x_hbm.at[src_v.at[c1 + 1]], rows_a, sem_ga)
            pltpu.sync_copy(rows_b, acc_sh.at[dst_v.at[c1]], add=True)
            return carry

        lax.fori_loop(0, (NCHUNK - 1) // 2, body, 0)

        pltpu.make_async_copy(x_hbm.at[src_v.at[NCHUNK - 1]], rows_a,
                              sem_ga).wait()
        pltpu.sync_copy(rows_a, acc_sh.at[dst_v.at[NCHUNK - 1]], add=True)

        plsc.subcore_barrier()

        pltpu.sync_copy(acc_sh.at[pl.ds(row0, ROWS_PER_TILE)],
                        acc_out.at[cid, pl.ds(row0, ROWS_PER_TILE)])

    return k(x, src_p, dst_p, z_acc)


HB = 2560
HI = 32
LO = 64


def _hist_body(dst_ref, o_ref):
    d = dst_ref[0]
    hi = jnp.broadcast_to(d >> 6, (HI, HB))
    lo = jnp.broadcast_to(d & 63, (LO, HB))
    ahi = (lax.broadcasted_iota(jnp.int32, (HI, HB), 0) == hi
           ).astype(jnp.float32)
    alo = (lax.broadcasted_iota(jnp.int32, (LO, HB), 0) == lo
           ).astype(jnp.float32)
    h = lax.dot_general(ahi, alo, (((1,), (1,)), ((), ())),
                        preferred_element_type=jnp.float32)

    @pl.when(pl.program_id(0) == 0)
    def _():
        o_ref[...] = h

    @pl.when(pl.program_id(0) != 0)
    def _():
        o_ref[...] += h


def _tc_body(acc_ref, cnt_ref, xt_ref, wlt_ref, wrt_ref, b_ref, o_ref):
    acc = acc_ref[0] + acc_ref[1]
    cnt = cnt_ref[...]
    agg = acc / jnp.maximum(cnt, 1.0)
    out = (jnp.dot(agg, wlt_ref[...], preferred_element_type=jnp.float32)
           + b_ref[...]
           + jnp.dot(xt_ref[...], wrt_ref[...],
                     preferred_element_type=jnp.float32))
    norm = jnp.sqrt(jnp.sum(out * out, axis=1, keepdims=True))
    o_ref[...] = out / jnp.maximum(norm, 1e-12)


def kernel(x, edge_index, W_l, W_r, b):
    src = edge_index[0].astype(jnp.int32)
    dst = edge_index[1].astype(jnp.int32)
    src_p = jnp.pad(src.reshape(NW, EDGES_PER_TILE),
                    ((0, 0), (0, SRC_ROWS * CHUNK - EDGES_PER_TILE))
                    ).reshape(NW, SRC_ROWS, CHUNK)
    npad = NCHUNK_PAD * CHUNK - EDGES_PER_TILE
    junk = jnp.broadcast_to(
        (N_DST + jnp.arange(NW, dtype=jnp.int32) % NS)[:, None], (NW, npad))
    dst_p = jnp.concatenate(
        [dst.reshape(NW, EDGES_PER_TILE), junk],
        axis=1).reshape(NW, NCHUNK_PAD, CHUNK)
    z_acc = jnp.zeros((N_DST, D), dtype=jnp.float32)

    acc_p = _sc_aggregate(x, src_p, dst_p, z_acc)

    cnt2d = pl.pallas_call(
        _hist_body,
        grid=(E // HB,),
        in_specs=[pl.BlockSpec((1, 1, HB), lambda i: (i, 0, 0))],
        out_specs=pl.BlockSpec((HI, LO), lambda i: (0, 0)),
        out_shape=jax.ShapeDtypeStruct((HI, LO), jnp.float32),
    )(dst.reshape(E // HB, 1, HB))
    cnt = cnt2d.reshape(N_DST, 1)

    out = pl.pallas_call(
        _tc_body,
        out_shape=jax.ShapeDtypeStruct((N_DST, D), jnp.float32),
    )(acc_p, cnt, x[:N_DST], W_l.T, W_r.T, b.reshape(1, D))
    return out
